# baseline (speedup 1.0000x reference)
import functools

import jax
import jax.numpy as jnp
from jax import lax
from jax.experimental import pallas as pl
from jax.experimental.pallas import tpu as pltpu
from jax.experimental.pallas import tpu_sc as plsc

B = 1024
CTX = 20
EMB = 64
VOCAB_ = 100000
LANES = 16


_HOFF = (0, 49920)
_HSIZE = (49920, 50048)
_HMAX = 50048
_TAIL_OFF = 99968
_TAIL = 32


def _make_pool_kernel():
    info = plsc.get_sparse_core_info()
    nc, ns = info.num_cores, info.num_subcores
    nw = nc * ns
    mesh = plsc.VectorSubcoreMesh(core_axis_name="c", subcore_axis_name="s")

    @functools.partial(
        pl.kernel,
        mesh=mesh,
        out_type=jax.ShapeDtypeStruct((EMB, B), jnp.float32),
        scratch_types=[
            pltpu.VMEM((CTX, B), jnp.int32),
            pltpu.VMEM((_HMAX,), jnp.float32),
            pltpu.VMEM((_HMAX,), jnp.float32),
            pltpu.VMEM((B,), jnp.float32),
            pltpu.VMEM((EMB * _TAIL,), jnp.float32),
            pltpu.SemaphoreType.DMA,
            pltpu.SemaphoreType.DMA,
        ],
        compiler_params=pltpu.CompilerParams(
            use_tc_tiling_on_sc=True, needs_layout_passes=False
        ),
    )
    def pool(
        idx_hbm, table_t_hbm, tail_hbm, out_hbm,
        idx_v, buf0, buf1, acc_v, tail_v, sem0, sem1,
    ):
        wid = lax.axis_index("s") * nc + lax.axis_index("c")
        e0 = wid * 2
        bufs = (buf0, buf1)
        sems = (sem0, sem1)

        def start_chunk(k):
            p, h = k // 2, k % 2
            return pltpu.async_copy(
                table_t_hbm.at[e0 + p, pl.ds(_HOFF[h], _HSIZE[h])],
                bufs[k % 2].at[pl.ds(0, _HSIZE[h])],
                sems[k % 2],
            )

        copy = start_chunk(0)
        pltpu.sync_copy(idx_hbm, idx_v)
        pltpu.sync_copy(tail_hbm, tail_v)
        for k in range(4):
            p, h = k // 2, k % 2
            copy.wait()
            if k + 1 < 4:
                nxt = start_chunk(k + 1)
            buf = bufs[k % 2]
            lo = _HOFF[h]
            hsize = _HSIZE[h]
            tail_base = (e0 + p) * _TAIL - _TAIL_OFF

            def group_body(g, carry, h=h, buf=buf, lo=lo, hsize=hsize,
                           tail_base=tail_base):
                base = g * LANES
                if h == 0:
                    acc = jnp.zeros((LANES,), jnp.float32)
                else:
                    acc = acc_v[pl.ds(base, LANES)]
                for j in range(CTX):
                    idx16 = idx_v[j, pl.ds(base, LANES)]
                    if h == 0:
                        mask = idx16 < lo + hsize
                    else:
                        mask = (idx16 >= lo) & (idx16 < lo + hsize)
                    off = jnp.clip(idx16 - lo, 0, hsize - 1)
                    gv = plsc.load_gather(buf, [off], mask=mask)
                    acc = acc + jnp.where(mask, gv, 0.0)
                    if h == 1:
                        tmask = idx16 >= _TAIL_OFF
                        toff = jnp.clip(idx16 + tail_base, 0, EMB * _TAIL - 1)
                        tv = plsc.load_gather(tail_v, [toff], mask=tmask)
                        acc = acc + jnp.where(tmask, tv, 0.0)
                if h == 1:
                    acc = acc * (1.0 / CTX)
                acc_v[pl.ds(base, LANES)] = acc
                return carry

            lax.fori_loop(0, B // LANES, group_body, 0)
            if h == 1:
                pltpu.sync_copy(acc_v, out_hbm.at[e0 + p])
            if k + 1 < 4:
                copy = nxt

    return pool


_BV = 4096


def _matmul_bias_t(pooled_t, linear_w, linear_b):
    v = linear_w.shape[1]
    nv = pl.cdiv(v, _BV)

    def mm(xt_ref, w_ref, b_ref, o_ref):
        wt_xt = lax.dot_general(
            w_ref[...],
            xt_ref[...],
            (((0,), (0,)), ((), ())),
            preferred_element_type=jnp.float32,
        )
        o_ref[...] = wt_xt + b_ref[...].T

    return pl.pallas_call(
        mm,
        grid=(nv,),
        in_specs=[
            pl.BlockSpec((EMB, B), lambda i: (0, 0)),
            pl.BlockSpec((EMB, _BV), lambda i: (0, i)),
            pl.BlockSpec((1, _BV), lambda i: (0, i)),
        ],
        out_specs=pl.BlockSpec((_BV, B), lambda i: (i, 0)),
        out_shape=jax.ShapeDtypeStruct((v, B), jnp.float32),
        compiler_params=pltpu.CompilerParams(
            dimension_semantics=("parallel",),
        ),
    )(pooled_t, linear_w, linear_b.reshape(1, v))


def kernel(context_idxs, embedding_table, linear_w, linear_b):
    idx_t = context_idxs.T.astype(jnp.int32)
    table_t = embedding_table.T
    tail = table_t[:, _TAIL_OFF:].reshape(-1)
    pooled_t = _make_pool_kernel()(idx_t, table_t, tail)
    return _matmul_bias_t(pooled_t, linear_w, linear_b).T


# speedup vs baseline: 1.0288x; 1.0288x over previous
import functools

import jax
import jax.numpy as jnp
from jax import lax
from jax.experimental import pallas as pl
from jax.experimental.pallas import tpu as pltpu
from jax.experimental.pallas import tpu_sc as plsc

B = 1024
CTX = 20
EMB = 64
VOCAB_ = 100000
LANES = 16


def _make_pool_kernel():
    info = plsc.get_sparse_core_info()
    nc, ns = info.num_cores, info.num_subcores
    nw = nc * ns
    dims_per_w = EMB // nw
    mesh = plsc.VectorSubcoreMesh(core_axis_name="c", subcore_axis_name="s")

    @functools.partial(
        pl.kernel,
        mesh=mesh,
        out_type=jax.ShapeDtypeStruct((EMB, B), jnp.float32),
        scratch_types=[
            pltpu.VMEM((CTX, B), jnp.int32),
            pltpu.VMEM((VOCAB_,), jnp.float32),
            pltpu.VMEM((B,), jnp.float32),
        ],
        compiler_params=pltpu.CompilerParams(
            use_tc_tiling_on_sc=True, needs_layout_passes=False
        ),
    )
    def pool(idx_hbm, table_t_hbm, out_hbm, idx_v, row_v, acc_v):
        wid = lax.axis_index("s") * nc + lax.axis_index("c")
        pltpu.sync_copy(idx_hbm, idx_v)
        for p in range(dims_per_w):
            e = wid * dims_per_w + p
            pltpu.sync_copy(table_t_hbm.at[e], row_v)

            def group_body(g, carry):
                base = g * LANES
                acc = jnp.zeros((LANES,), jnp.float32)
                for j in range(CTX):
                    idx16 = idx_v[j, pl.ds(base, LANES)]
                    acc = acc + plsc.load_gather(row_v, [idx16])
                acc_v[pl.ds(base, LANES)] = acc * (1.0 / CTX)
                return carry

            lax.fori_loop(0, B // LANES, group_body, 0)
            pltpu.sync_copy(acc_v, out_hbm.at[e])

    return pool


_BV = 4096


def _matmul_bias_t(pooled_t, linear_w, linear_b):
    v = linear_w.shape[1]
    nv = pl.cdiv(v, _BV)

    def mm(xt_ref, w_ref, b_ref, o_ref):
        wt_xt = lax.dot_general(
            w_ref[...],
            xt_ref[...],
            (((0,), (0,)), ((), ())),
            preferred_element_type=jnp.float32,
        )
        o_ref[...] = wt_xt + b_ref[...].T

    return pl.pallas_call(
        mm,
        grid=(nv,),
        in_specs=[
            pl.BlockSpec((EMB, B), lambda i: (0, 0)),
            pl.BlockSpec((EMB, _BV), lambda i: (0, i)),
            pl.BlockSpec((1, _BV), lambda i: (0, i)),
        ],
        out_specs=pl.BlockSpec((_BV, B), lambda i: (i, 0)),
        out_shape=jax.ShapeDtypeStruct((v, B), jnp.float32),
        compiler_params=pltpu.CompilerParams(
            dimension_semantics=("parallel",),
        ),
    )(pooled_t, linear_w, linear_b.reshape(1, v))


def kernel(context_idxs, embedding_table, linear_w, linear_b):
    idx_t = context_idxs.T.astype(jnp.int32)
    pooled_t = _make_pool_kernel()(idx_t, embedding_table.T)
    return _matmul_bias_t(pooled_t, linear_w, linear_b).T
